# Initial kernel scaffold; baseline (speedup 1.0000x reference)
import functools

import jax
import jax.numpy as jnp
from jax import lax
from jax.experimental import pallas as pl
from jax.experimental.pallas import tpu as pltpu
from jax.experimental.pallas import tpu_sc as plsc


def _proj_body(table_ref, w1_ref, p_ref):
    p_ref[...] = jnp.dot(table_ref[...], w1_ref[...],
                         preferred_element_type=jnp.float32,
                         precision=lax.Precision.HIGHEST)


def _project_table(table, W1):
    n, d = table.shape
    h = W1.shape[1]
    return pl.pallas_call(
        _proj_body,
        out_shape=jax.ShapeDtypeStruct((n, h), jnp.float32),
    )(table, W1)


def _make_sc_gather(E, H, n_workers, block):
    epw = E // n_workers
    n_blocks = epw // block
    mesh = plsc.VectorSubcoreMesh(core_axis_name="c", subcore_axis_name="s")
    nc = mesh.num_cores

    @functools.partial(
        pl.kernel,
        out_type=jax.ShapeDtypeStruct((E, H), jnp.float32),
        mesh=mesh,
        scratch_types=[
            pltpu.VMEM((epw,), jnp.int32),
            pltpu.VMEM((epw,), jnp.int32),
            pltpu.VMEM((block, H), jnp.float32),
            pltpu.SemaphoreType.DMA,
        ],
    )
    def sc_gather(p_hbm, src_hbm, dst_hbm, g_hbm, sidx, didx, buf, sem):
        wid = lax.axis_index("s") * nc + lax.axis_index("c")
        base = wid * epw
        pltpu.sync_copy(src_hbm.at[pl.ds(base, epw)], sidx)
        pltpu.sync_copy(dst_hbm.at[pl.ds(base, epw)], didx)

        def body(j, carry):
            off = j * block
            pltpu.async_copy(p_hbm.at[sidx.at[pl.ds(off, block)]], buf,
                             sem).wait()
            pltpu.async_copy(p_hbm.at[didx.at[pl.ds(off, block)]], buf,
                             sem, add=True).wait()
            pltpu.sync_copy(buf, g_hbm.at[pl.ds(base + off, block)])
            return carry

        lax.fori_loop(0, n_blocks, body, 0, unroll=False)

    return sc_gather


def _mlp_body(g_ref, w2_ref, b1_ref, b2_ref, out_ref):
    h = jnp.maximum(g_ref[...] + b1_ref[...], 0.0)
    o = jnp.dot(h, w2_ref[...], preferred_element_type=jnp.float32,
                precision=lax.Precision.HIGHEST)
    o = jnp.maximum(o + b2_ref[...], 0.0)
    m = jnp.max(o, axis=1, keepdims=True)
    ex = jnp.exp(o - m)
    out_ref[...] = ex / jnp.sum(ex, axis=1, keepdims=True)


def _mlp_tail(g, W2, b1, b2, block_e):
    e, h = g.shape
    c1 = W2.shape[1]
    grid = e // block_e
    return pl.pallas_call(
        _mlp_body,
        grid=(grid,),
        in_specs=[
            pl.BlockSpec((block_e, h), lambda i: (i, 0)),
            pl.BlockSpec((h, c1), lambda i: (0, 0)),
            pl.BlockSpec((1, h), lambda i: (0, 0)),
            pl.BlockSpec((1, c1), lambda i: (0, 0)),
        ],
        out_specs=pl.BlockSpec((block_e, c1), lambda i: (i, 0)),
        out_shape=jax.ShapeDtypeStruct((e, c1), jnp.float32),
    )(g, W2, b1.reshape(1, h), b2.reshape(1, c1))


@jax.jit
def kernel(src, dst, table, W1, b1, W2, b2):
    E = src.shape[0]
    H = W1.shape[1]
    P = _project_table(table, W1)
    g = _make_sc_gather(E, H, n_workers=32, block=80)(P, src, dst)
    return _mlp_tail(g, W2, b1, b2, block_e=2000)


# speedup vs baseline: 2.4833x; 2.4833x over previous
import functools

import jax
import jax.numpy as jnp
from jax import lax
from jax.experimental import pallas as pl
from jax.experimental.pallas import tpu as pltpu
from jax.experimental.pallas import tpu_sc as plsc


def _proj_body(table_ref, w1_ref, p_ref):
    p_ref[...] = jnp.dot(table_ref[...], w1_ref[...],
                         preferred_element_type=jnp.float32,
                         precision=lax.Precision.HIGHEST)


def _project_table(table, W1):
    n, d = table.shape
    h = W1.shape[1]
    return pl.pallas_call(
        _proj_body,
        out_shape=jax.ShapeDtypeStruct((n, h), jnp.float32),
    )(table, W1)


def _make_sc_gather(E, H, n_workers, block):
    epw = E // n_workers
    n_blocks = epw // block
    mesh = plsc.VectorSubcoreMesh(core_axis_name="c", subcore_axis_name="s")
    nc = mesh.num_cores

    @functools.partial(
        pl.kernel,
        out_type=jax.ShapeDtypeStruct((E, H), jnp.float32),
        mesh=mesh,
        scratch_types=[
            pltpu.VMEM((epw,), jnp.int32),
            pltpu.VMEM((epw,), jnp.int32),
            pltpu.VMEM((block, H), jnp.float32),
            pltpu.SemaphoreType.DMA,
        ],
        compiler_params=pltpu.CompilerParams(use_tc_tiling_on_sc=False),
    )
    def sc_gather(p_hbm, src_hbm, dst_hbm, g_hbm, sidx, didx, buf, sem):
        wid = lax.axis_index("s") * nc + lax.axis_index("c")
        base = wid * epw
        pltpu.sync_copy(src_hbm.at[pl.ds(base, epw)], sidx)
        pltpu.sync_copy(dst_hbm.at[pl.ds(base, epw)], didx)

        def body(j, carry):
            off = j * block
            pltpu.async_copy(p_hbm.at[sidx.at[pl.ds(off, block)]], buf,
                             sem).wait()
            pltpu.async_copy(p_hbm.at[didx.at[pl.ds(off, block)]], buf,
                             sem, add=True).wait()
            pltpu.sync_copy(buf, g_hbm.at[pl.ds(base + off, block)])
            return carry

        lax.fori_loop(0, n_blocks, body, 0, unroll=False)

    return sc_gather


def _mlp_body(g_ref, w2_ref, b1_ref, b2_ref, out_ref):
    h = jnp.maximum(g_ref[...] + b1_ref[...], 0.0)
    o = jnp.dot(h, w2_ref[...], preferred_element_type=jnp.float32,
                precision=lax.Precision.HIGHEST)
    o = jnp.maximum(o + b2_ref[...], 0.0)
    m = jnp.max(o, axis=1, keepdims=True)
    ex = jnp.exp(o - m)
    out_ref[...] = ex / jnp.sum(ex, axis=1, keepdims=True)


def _mlp_tail(g, W2, b1, b2, block_e):
    e, h = g.shape
    c1 = W2.shape[1]
    grid = e // block_e
    return pl.pallas_call(
        _mlp_body,
        grid=(grid,),
        in_specs=[
            pl.BlockSpec((block_e, h), lambda i: (i, 0)),
            pl.BlockSpec((h, c1), lambda i: (0, 0)),
            pl.BlockSpec((1, h), lambda i: (0, 0)),
            pl.BlockSpec((1, c1), lambda i: (0, 0)),
        ],
        out_specs=pl.BlockSpec((block_e, c1), lambda i: (i, 0)),
        out_shape=jax.ShapeDtypeStruct((e, c1), jnp.float32),
    )(g, W2, b1.reshape(1, h), b2.reshape(1, c1))


@jax.jit
def kernel(src, dst, table, W1, b1, W2, b2):
    E = src.shape[0]
    H = W1.shape[1]
    P = _project_table(table, W1)
    g = _make_sc_gather(E, H, n_workers=32, block=80)(P, src, dst)
    return _mlp_tail(g, W2, b1, b2, block_e=2000)


# speedup vs baseline: 4.1374x; 1.6661x over previous
import functools

import jax
import jax.numpy as jnp
from jax import lax
from jax.experimental import pallas as pl
from jax.experimental.pallas import tpu as pltpu
from jax.experimental.pallas import tpu_sc as plsc


def _proj_body(table_ref, w1_ref, p_ref):
    p_ref[...] = jnp.dot(table_ref[...], w1_ref[...],
                         preferred_element_type=jnp.float32,
                         precision=lax.Precision.HIGHEST)


def _project_table(table, W1):
    n, d = table.shape
    h = W1.shape[1]
    return pl.pallas_call(
        _proj_body,
        out_shape=jax.ShapeDtypeStruct((n, h), jnp.float32),
    )(table, W1)


def _make_sc_gather(E, H, n_workers, block):
    epw = E // n_workers
    n_pairs = epw // (2 * block)
    tail = epw - n_pairs * 2 * block
    mesh = plsc.VectorSubcoreMesh(core_axis_name="c", subcore_axis_name="s")
    nc = mesh.num_cores

    @functools.partial(
        pl.kernel,
        out_type=jax.ShapeDtypeStruct((E, H), jnp.float32),
        mesh=mesh,
        scratch_types=[
            pltpu.VMEM((epw,), jnp.int32),
            pltpu.VMEM((epw,), jnp.int32),
            pltpu.VMEM((block, H), jnp.float32),
            pltpu.VMEM((block, H), jnp.float32),
            pltpu.SemaphoreType.DMA,
            pltpu.SemaphoreType.DMA,
        ],
        compiler_params=pltpu.CompilerParams(use_tc_tiling_on_sc=False),
    )
    def sc_gather(p_hbm, src_hbm, dst_hbm, g_hbm, sidx, didx, buf_a, buf_b,
                  sem_a, sem_b):
        wid = lax.axis_index("s") * nc + lax.axis_index("c")
        base = wid * epw
        pltpu.sync_copy(src_hbm.at[pl.ds(base, epw)], sidx)
        pltpu.sync_copy(dst_hbm.at[pl.ds(base, epw)], didx)

        def chunk(off, n, buf, sem):
            gs = pltpu.async_copy(p_hbm.at[sidx.at[pl.ds(off, n)]],
                                  buf.at[pl.ds(0, n)], sem)
            return gs

        def body(j2, carry):
            oa = j2 * (2 * block)
            ob = oa + block
            gs_a = pltpu.async_copy(p_hbm.at[sidx.at[pl.ds(oa, block)]],
                                    buf_a, sem_a)
            gs_b = pltpu.async_copy(p_hbm.at[sidx.at[pl.ds(ob, block)]],
                                    buf_b, sem_b)
            gs_a.wait()
            gd_a = pltpu.async_copy(p_hbm.at[didx.at[pl.ds(oa, block)]],
                                    buf_a, sem_a, add=True)
            gs_b.wait()
            gd_b = pltpu.async_copy(p_hbm.at[didx.at[pl.ds(ob, block)]],
                                    buf_b, sem_b, add=True)
            gd_a.wait()
            st_a = pltpu.async_copy(buf_a, g_hbm.at[pl.ds(base + oa, block)],
                                    sem_a)
            gd_b.wait()
            st_b = pltpu.async_copy(buf_b, g_hbm.at[pl.ds(base + ob, block)],
                                    sem_b)
            st_a.wait()
            st_b.wait()
            return carry

        lax.fori_loop(0, n_pairs, body, 0, unroll=False)

        if tail:
            off = n_pairs * 2 * block
            pltpu.async_copy(p_hbm.at[sidx.at[pl.ds(off, tail)]],
                             buf_a.at[pl.ds(0, tail)], sem_a).wait()
            pltpu.async_copy(p_hbm.at[didx.at[pl.ds(off, tail)]],
                             buf_a.at[pl.ds(0, tail)], sem_a, add=True).wait()
            pltpu.async_copy(buf_a.at[pl.ds(0, tail)],
                             g_hbm.at[pl.ds(base + off, tail)], sem_a).wait()

    return sc_gather


def _mlp_body(g_ref, w2_ref, b1_ref, b2_ref, out_ref):
    h = jnp.maximum(g_ref[...] + b1_ref[...], 0.0)
    o = lax.dot_general(w2_ref[...], h, (((0,), (1,)), ((), ())),
                        preferred_element_type=jnp.float32,
                        precision=lax.Precision.HIGHEST)
    o = jnp.maximum(o + b2_ref[...], 0.0)
    m = jnp.max(o, axis=0, keepdims=True)
    ex = jnp.exp(o - m)
    out_ref[...] = ex / jnp.sum(ex, axis=0, keepdims=True)


def _mlp_tail(g, W2, b1, b2, block_e):
    e, h = g.shape
    c1 = W2.shape[1]
    grid = e // block_e
    outT = pl.pallas_call(
        _mlp_body,
        grid=(grid,),
        in_specs=[
            pl.BlockSpec((block_e, h), lambda i: (i, 0)),
            pl.BlockSpec((h, c1), lambda i: (0, 0)),
            pl.BlockSpec((1, h), lambda i: (0, 0)),
            pl.BlockSpec((c1, 1), lambda i: (0, 0)),
        ],
        out_specs=pl.BlockSpec((c1, block_e), lambda i: (0, i)),
        out_shape=jax.ShapeDtypeStruct((c1, e), jnp.float32),
    )(g, W2, b1.reshape(1, h), b2.reshape(c1, 1))
    return outT.T


@jax.jit
def kernel(src, dst, table, W1, b1, W2, b2):
    E = src.shape[0]
    H = W1.shape[1]
    P = _project_table(table, W1)
    g = _make_sc_gather(E, H, n_workers=32, block=128)(P, src, dst)
    return _mlp_tail(g, W2, b1, b2, block_e=6400)
